# baseline (speedup 1.0000x reference)
import functools

import jax
import jax.numpy as jnp
from jax import lax
from jax.experimental import pallas as pl
from jax.experimental.pallas import tpu as pltpu
from jax.experimental.pallas import tpu_sc as plsc

N = 10000
NPAD = 10240
E = 160000
EPAD = 163840
CHUNK = 128
N_SC = 2
N_TILE = 16
ROWS_PER_TILE = NPAD // N_TILE
DEG_CHUNKS = EPAD // (N_SC * N_TILE * CHUNK)
AGG_CHUNKS = EPAD // (N_TILE * CHUNK)


def _sc_mesh():
    return plsc.VectorSubcoreMesh(core_axis_name="c", subcore_axis_name="s")


_SC_PARAMS = pltpu.CompilerParams(use_tc_tiling_on_sc=False)


def _deg(dsts, ones_h, zeros_h):
    def body(dst_h, ones_hbm, zeros_hbm, out, ones_v, dst_v, acc):
        cid = lax.axis_index("c")
        sid = lax.axis_index("s")
        pltpu.sync_copy(dst_h.at[cid, sid], dst_v)
        pltpu.sync_copy(ones_hbm, ones_v)
        r0 = sid * ROWS_PER_TILE
        pltpu.sync_copy(zeros_hbm, acc.at[pl.ds(r0, ROWS_PER_TILE)])
        plsc.subcore_barrier()

        def step(j, carry):
            pltpu.sync_copy(ones_v, acc.at[dst_v.at[j]], add=True)
            return carry

        lax.fori_loop(0, DEG_CHUNKS, step, 0)
        plsc.subcore_barrier()
        pltpu.sync_copy(acc.at[pl.ds(r0, ROWS_PER_TILE)],
                        out.at[cid, pl.ds(r0, ROWS_PER_TILE)])

    kfn = pl.kernel(
        body,
        out_type=jax.ShapeDtypeStruct((N_SC, NPAD, 16), jnp.float32),
        mesh=_sc_mesh(),
        scratch_types=[
            pltpu.VMEM((CHUNK, 16), jnp.float32),
            pltpu.VMEM((DEG_CHUNKS, CHUNK), jnp.int32),
            pltpu.VMEM_SHARED((NPAD, 16), jnp.float32),
        ],
        compiler_params=_SC_PARAMS,
    )
    return kfn(dsts, ones_h, zeros_h)


def _agg(g_cat, srcs, dsts, feat):
    n_pass = 2 if feat > 64 else 1
    chunks_per_pass = AGG_CHUNKS // n_pass

    def body(g_h, src_h, dst_h, out, src_v, dst_v, buf0, buf1, acc,
             gs0, gs1, ss0, ss1):
        cid = lax.axis_index("c")
        sid = lax.axis_index("s")
        r0 = sid * ROWS_PER_TILE
        pltpu.sync_copy(g_h.at[pl.ds(cid * NPAD + r0, ROWS_PER_TILE)],
                        acc.at[pl.ds(r0, ROWS_PER_TILE)])
        plsc.subcore_barrier()

        bufs = (buf0, buf1)
        gsems = (gs0, gs1)
        ssems = (ss0, ss1)
        n = chunks_per_pass

        for p in range(n_pass):
            c0 = p * n
            pltpu.sync_copy(src_h.at[cid, sid, pl.ds(c0, n)], src_v)
            pltpu.sync_copy(dst_h.at[sid, pl.ds(c0, n)], dst_v)

            pltpu.async_copy(g_h.at[src_v.at[0]], buf0, gsems[0])

            def step(j, carry):
                for b in range(2):
                    @pl.when(lax.rem(j, 2) == b)
                    def _():
                        bn = (b + 1) % 2
                        @pl.when(j + 1 < n)
                        def _():
                            @pl.when(j >= 1)
                            def _():
                                pltpu.make_async_copy(
                                    bufs[bn], acc.at[dst_v.at[j]],
                                    ssems[bn]).wait()

                            pltpu.async_copy(g_h.at[src_v.at[j + 1]],
                                             bufs[bn], gsems[bn])

                        pltpu.make_async_copy(
                            g_h.at[src_v.at[j]], bufs[b], gsems[b]).wait()
                        pltpu.async_copy(bufs[b], acc.at[dst_v.at[j]],
                                         ssems[b], add=True)

                return carry

            lax.fori_loop(0, n, step, 0)
            for t in range(2):
                j = n - 1 - t
                pltpu.make_async_copy(bufs[j % 2], acc.at[dst_v.at[0]],
                                      ssems[j % 2]).wait()
        plsc.subcore_barrier()
        pltpu.sync_copy(acc.at[pl.ds(r0, ROWS_PER_TILE)],
                        out.at[cid, pl.ds(r0, ROWS_PER_TILE)])

    kfn = pl.kernel(
        body,
        out_type=jax.ShapeDtypeStruct((N_SC, NPAD, feat), jnp.float32),
        mesh=_sc_mesh(),
        scratch_types=[
            pltpu.VMEM((AGG_CHUNKS // n_pass, CHUNK), jnp.int32),
            pltpu.VMEM((AGG_CHUNKS // n_pass, CHUNK), jnp.int32),
            pltpu.VMEM((CHUNK, feat), jnp.float32),
            pltpu.VMEM((CHUNK, feat), jnp.float32),
            pltpu.VMEM_SHARED((NPAD, feat), jnp.float32),
            pltpu.SemaphoreType.DMA,
            pltpu.SemaphoreType.DMA,
            pltpu.SemaphoreType.DMA,
            pltpu.SemaphoreType.DMA,
        ],
        compiler_params=_SC_PARAMS,
    )
    return kfn(g_cat, srcs, dsts)


AGG2_CHUNKS = EPAD // (N_SC * N_TILE * CHUNK)


def _agg2(g2, srcs, dsts, zeros_h):
    def body(g_h, src_h, dst_h, z_h, out, src_v, dst_v, buf0, buf1, acc,
             gs0, gs1, ss0, ss1):
        cid = lax.axis_index("c")
        sid = lax.axis_index("s")
        r0 = sid * ROWS_PER_TILE
        pltpu.sync_copy(src_h.at[cid, sid], src_v)
        pltpu.sync_copy(dst_h.at[cid, sid], dst_v)

        @pl.when(cid == 0)
        def _():
            pltpu.sync_copy(g_h.at[pl.ds(r0, ROWS_PER_TILE)],
                            acc.at[pl.ds(r0, ROWS_PER_TILE)])

        @pl.when(cid == 1)
        def _():
            pltpu.sync_copy(z_h, acc.at[pl.ds(r0, ROWS_PER_TILE)])

        plsc.subcore_barrier()

        bufs = (buf0, buf1)
        gsems = (gs0, gs1)
        ssems = (ss0, ss1)
        n = AGG2_CHUNKS

        pltpu.async_copy(g_h.at[src_v.at[0]], buf0, gsems[0])

        def step(j, carry):
            for b in range(2):
                @pl.when(lax.rem(j, 2) == b)
                def _():
                    bn = (b + 1) % 2
                    @pl.when(j + 1 < n)
                    def _():
                        @pl.when(j >= 1)
                        def _():
                            pltpu.make_async_copy(
                                bufs[bn], acc.at[dst_v.at[j]],
                                ssems[bn]).wait()

                        pltpu.async_copy(g_h.at[src_v.at[j + 1]],
                                         bufs[bn], gsems[bn])

                    pltpu.make_async_copy(
                        g_h.at[src_v.at[j]], bufs[b], gsems[b]).wait()
                    pltpu.async_copy(bufs[b], acc.at[dst_v.at[j]],
                                     ssems[b], add=True)

            return carry

        lax.fori_loop(0, n, step, 0)
        for t in range(2):
            j = n - 1 - t
            pltpu.make_async_copy(bufs[j % 2], acc.at[dst_v.at[0]],
                                  ssems[j % 2]).wait()
        plsc.subcore_barrier()
        pltpu.sync_copy(acc.at[pl.ds(r0, ROWS_PER_TILE)],
                        out.at[cid, pl.ds(r0, ROWS_PER_TILE)])

    kfn = pl.kernel(
        body,
        out_type=jax.ShapeDtypeStruct((N_SC, NPAD, 64), jnp.float32),
        mesh=_sc_mesh(),
        scratch_types=[
            pltpu.VMEM((AGG2_CHUNKS, CHUNK), jnp.int32),
            pltpu.VMEM((AGG2_CHUNKS, CHUNK), jnp.int32),
            pltpu.VMEM((CHUNK, 64), jnp.float32),
            pltpu.VMEM((CHUNK, 64), jnp.float32),
            pltpu.VMEM_SHARED((NPAD, 64), jnp.float32),
            pltpu.SemaphoreType.DMA,
            pltpu.SemaphoreType.DMA,
            pltpu.SemaphoreType.DMA,
            pltpu.SemaphoreType.DMA,
        ],
        compiler_params=_SC_PARAMS,
    )
    return kfn(g2, srcs, dsts, zeros_h)


def _mm1(x_pad, W1):
    def body(x_ref, w_ref, o_ref):
        o_ref[...] = jnp.dot(x_ref[...], w_ref[...],
                             preferred_element_type=jnp.float32)

    return pl.pallas_call(
        body,
        grid=(NPAD // 512,),
        in_specs=[pl.BlockSpec((512, 256), lambda i: (i, 0)),
                  pl.BlockSpec((256, 256), lambda i: (0, 0))],
        out_specs=pl.BlockSpec((512, 256), lambda i: (i, 0)),
        out_shape=jax.ShapeDtypeStruct((NPAD, 256), jnp.float32),
    )(x_pad, W1)


def _dinv_of(d_blk):
    deg = d_blk[0, :, 0:1] + d_blk[1, :, 0:1] + 1.0
    return lax.rsqrt(deg)


def _scale_split(h1, deg_parts):
    bm = 256

    def body(h_ref, d_ref, g_ref):
        g = h_ref[...] * _dinv_of(d_ref)
        g_ref[0] = g[:, :128]
        g_ref[1] = g[:, 128:]

    return pl.pallas_call(
        body,
        grid=(NPAD // bm,),
        in_specs=[pl.BlockSpec((bm, 256), lambda i: (i, 0)),
                  pl.BlockSpec((2, bm, 16), lambda i: (0, i, 0))],
        out_specs=pl.BlockSpec((2, bm, 128), lambda i: (0, i, 0)),
        out_shape=jax.ShapeDtypeStruct((2, NPAD, 128), jnp.float32),
    )(h1, deg_parts)


def _layer2(agg1, deg_parts, b1r, W2):
    bm = 256

    def body(a_ref, d_ref, b1_ref, w2_ref, g2_ref):
        dinv = _dinv_of(d_ref)
        z1 = jnp.concatenate([a_ref[0], a_ref[1]], axis=1)
        z1 = jax.nn.relu(z1 * dinv + b1_ref[...])
        h2 = jnp.dot(z1, w2_ref[...], preferred_element_type=jnp.float32)
        g2_ref[...] = h2 * dinv

    return pl.pallas_call(
        body,
        grid=(NPAD // bm,),
        in_specs=[pl.BlockSpec((2, bm, 128), lambda i: (0, i, 0)),
                  pl.BlockSpec((2, bm, 16), lambda i: (0, i, 0)),
                  pl.BlockSpec((1, 256), lambda i: (0, 0)),
                  pl.BlockSpec((256, 64), lambda i: (0, 0))],
        out_specs=pl.BlockSpec((bm, 64), lambda i: (i, 0)),
        out_shape=jax.ShapeDtypeStruct((NPAD, 64), jnp.float32),
    )(agg1, deg_parts, b1r, W2)


def _decoder(parts, deg_parts, b2r):
    bm = 1024

    def body(pa_ref, pb_ref, da_ref, db_ref, b2_ref, o_ref, z_ref):
        za = (pa_ref[0] + pa_ref[1]) * _dinv_of(da_ref) + b2_ref[...]
        zb = (pb_ref[0] + pb_ref[1]) * _dinv_of(db_ref) + b2_ref[...]
        prod = lax.dot_general(za, zb, (((1,), (1,)), ((), ())),
                               preferred_element_type=jnp.float32)
        o_ref[...] = jax.nn.sigmoid(prod)
        z_ref[...] = za

    return pl.pallas_call(
        body,
        grid=(NPAD // bm, NPAD // bm),
        in_specs=[pl.BlockSpec((2, bm, 64), lambda i, j: (0, i, 0)),
                  pl.BlockSpec((2, bm, 64), lambda i, j: (0, j, 0)),
                  pl.BlockSpec((2, bm, 16), lambda i, j: (0, i, 0)),
                  pl.BlockSpec((2, bm, 16), lambda i, j: (0, j, 0)),
                  pl.BlockSpec((1, 64), lambda i, j: (0, 0))],
        out_specs=[pl.BlockSpec((bm, bm), lambda i, j: (i, j)),
                   pl.BlockSpec((bm, 64), lambda i, j: (i, 0))],
        out_shape=[jax.ShapeDtypeStruct((N, N), jnp.float32),
                   jax.ShapeDtypeStruct((NPAD, 64), jnp.float32)],
    )(parts, parts, deg_parts, deg_parts, b2r)


def kernel(x, edge_index, W1, b1, W2, b2):
    src = edge_index[0].astype(jnp.int32)
    dst = edge_index[1].astype(jnp.int32)
    pad = EPAD - E
    src_p = jnp.concatenate([src, jnp.full((pad,), N, jnp.int32)])
    dst_p = jnp.concatenate([dst, jnp.full((pad,), N, jnp.int32)])

    dst_deg = dst_p.reshape(N_SC, N_TILE, DEG_CHUNKS, CHUNK)
    src_t = src_p.reshape(N_TILE, AGG_CHUNKS, CHUNK)
    srcs_agg = jnp.stack([src_t, src_t + NPAD])
    dst_agg = dst_p.reshape(N_TILE, AGG_CHUNKS, CHUNK)
    srcs2 = src_p.reshape(N_SC, N_TILE, AGG2_CHUNKS, CHUNK)
    dsts2 = dst_p.reshape(N_SC, N_TILE, AGG2_CHUNKS, CHUNK)

    x_pad = jnp.pad(x, ((0, NPAD - N), (0, 0)))
    ones_h = jnp.ones((CHUNK, 16), jnp.float32)
    zeros_h = jnp.zeros((ROWS_PER_TILE, 16), jnp.float32)
    zeros64_h = jnp.zeros((ROWS_PER_TILE, 64), jnp.float32)

    deg_parts = _deg(dst_deg, ones_h, zeros_h)
    h1 = _mm1(x_pad, W1)
    g1_cat = _scale_split(h1, deg_parts)
    agg1 = _agg(g1_cat.reshape(N_SC * NPAD, 128), srcs_agg, dst_agg, 128)
    g2 = _layer2(agg1, deg_parts, b1.reshape(1, 256), W2)
    parts = _agg2(g2, srcs2, dsts2, zeros64_h)
    recons, z_pad = _decoder(parts, deg_parts, b2.reshape(1, 64))
    return (recons, z_pad[:N])


# speedup vs baseline: 1.0571x; 1.0571x over previous
import functools

import jax
import jax.numpy as jnp
from jax import lax
from jax.experimental import pallas as pl
from jax.experimental.pallas import tpu as pltpu
from jax.experimental.pallas import tpu_sc as plsc

N = 10000
NPAD = 10240
E = 160000
EPAD = 163840
CHUNK = 128
N_SC = 2
N_TILE = 16
ROWS_PER_TILE = NPAD // N_TILE
DEG_CHUNKS = EPAD // (N_SC * N_TILE * CHUNK)
AGG_CHUNKS = EPAD // (N_TILE * CHUNK)


def _sc_mesh():
    return plsc.VectorSubcoreMesh(core_axis_name="c", subcore_axis_name="s")


_SC_PARAMS = pltpu.CompilerParams(use_tc_tiling_on_sc=False)


def _deg(dsts, ones_h, zeros_h):
    def body(dst_h, ones_hbm, zeros_hbm, out, ones_v, dst_v, acc):
        cid = lax.axis_index("c")
        sid = lax.axis_index("s")
        pltpu.sync_copy(dst_h.at[cid, sid], dst_v)
        pltpu.sync_copy(ones_hbm, ones_v)
        r0 = sid * ROWS_PER_TILE
        pltpu.sync_copy(zeros_hbm, acc.at[pl.ds(r0, ROWS_PER_TILE)])
        plsc.subcore_barrier()

        def step(j, carry):
            pltpu.sync_copy(ones_v, acc.at[dst_v.at[j]], add=True)
            return carry

        lax.fori_loop(0, DEG_CHUNKS, step, 0)
        plsc.subcore_barrier()
        pltpu.sync_copy(acc.at[pl.ds(r0, ROWS_PER_TILE)],
                        out.at[cid, pl.ds(r0, ROWS_PER_TILE)])

    kfn = pl.kernel(
        body,
        out_type=jax.ShapeDtypeStruct((N_SC, NPAD, 16), jnp.float32),
        mesh=_sc_mesh(),
        scratch_types=[
            pltpu.VMEM((CHUNK, 16), jnp.float32),
            pltpu.VMEM((DEG_CHUNKS, CHUNK), jnp.int32),
            pltpu.VMEM_SHARED((NPAD, 16), jnp.float32),
        ],
        compiler_params=_SC_PARAMS,
    )
    return kfn(dsts, ones_h, zeros_h)


def _agg(g_cat, srcs, dsts, feat):
    n_pass = 2 if feat > 64 else 1
    chunks_per_pass = AGG_CHUNKS // n_pass

    def body(g_h, src_h, dst_h, out, src_v, dst_v, buf0, buf1, acc,
             gs0, gs1, ss0, ss1):
        cid = lax.axis_index("c")
        sid = lax.axis_index("s")
        r0 = sid * ROWS_PER_TILE
        pltpu.sync_copy(g_h.at[pl.ds(cid * NPAD + r0, ROWS_PER_TILE)],
                        acc.at[pl.ds(r0, ROWS_PER_TILE)])
        plsc.subcore_barrier()

        bufs = (buf0, buf1)
        gsems = (gs0, gs1)
        ssems = (ss0, ss1)
        n = chunks_per_pass

        for p in range(n_pass):
            c0 = p * n
            pltpu.sync_copy(src_h.at[cid, sid, pl.ds(c0, n)], src_v)
            pltpu.sync_copy(dst_h.at[sid, pl.ds(c0, n)], dst_v)

            pltpu.async_copy(g_h.at[src_v.at[0]], buf0, gsems[0])

            def step(j, carry):
                for b in range(2):
                    @pl.when(lax.rem(j, 2) == b)
                    def _():
                        bn = (b + 1) % 2
                        @pl.when(j + 1 < n)
                        def _():
                            @pl.when(j >= 1)
                            def _():
                                pltpu.make_async_copy(
                                    bufs[bn], acc.at[dst_v.at[j]],
                                    ssems[bn]).wait()

                            pltpu.async_copy(g_h.at[src_v.at[j + 1]],
                                             bufs[bn], gsems[bn])

                        pltpu.make_async_copy(
                            g_h.at[src_v.at[j]], bufs[b], gsems[b]).wait()
                        pltpu.async_copy(bufs[b], acc.at[dst_v.at[j]],
                                         ssems[b], add=True)

                return carry

            lax.fori_loop(0, n, step, 0)
            for t in range(2):
                j = n - 1 - t
                pltpu.make_async_copy(bufs[j % 2], acc.at[dst_v.at[0]],
                                      ssems[j % 2]).wait()
        plsc.subcore_barrier()
        pltpu.sync_copy(acc.at[pl.ds(r0, ROWS_PER_TILE)],
                        out.at[cid, pl.ds(r0, ROWS_PER_TILE)])

    kfn = pl.kernel(
        body,
        out_type=jax.ShapeDtypeStruct((N_SC, NPAD, feat), jnp.float32),
        mesh=_sc_mesh(),
        scratch_types=[
            pltpu.VMEM((AGG_CHUNKS // n_pass, CHUNK), jnp.int32),
            pltpu.VMEM((AGG_CHUNKS // n_pass, CHUNK), jnp.int32),
            pltpu.VMEM((CHUNK, feat), jnp.float32),
            pltpu.VMEM((CHUNK, feat), jnp.float32),
            pltpu.VMEM_SHARED((NPAD, feat), jnp.float32),
            pltpu.SemaphoreType.DMA,
            pltpu.SemaphoreType.DMA,
            pltpu.SemaphoreType.DMA,
            pltpu.SemaphoreType.DMA,
        ],
        compiler_params=_SC_PARAMS,
    )
    return kfn(g_cat, srcs, dsts)


AGG2_CHUNKS = EPAD // (N_SC * N_TILE * CHUNK)


def _agg2(g2, srcs, dsts, zeros_h):
    def body(g_h, src_h, dst_h, z_h, out, src_v, dst_v, buf0, buf1, acc,
             gs0, gs1, ss0, ss1):
        cid = lax.axis_index("c")
        sid = lax.axis_index("s")
        r0 = sid * ROWS_PER_TILE
        pltpu.sync_copy(src_h.at[cid, sid], src_v)
        pltpu.sync_copy(dst_h.at[cid, sid], dst_v)

        @pl.when(cid == 0)
        def _():
            pltpu.sync_copy(g_h.at[pl.ds(r0, ROWS_PER_TILE)],
                            acc.at[pl.ds(r0, ROWS_PER_TILE)])

        @pl.when(cid == 1)
        def _():
            pltpu.sync_copy(z_h, acc.at[pl.ds(r0, ROWS_PER_TILE)])

        plsc.subcore_barrier()

        bufs = (buf0, buf1)
        gsems = (gs0, gs1)
        ssems = (ss0, ss1)
        n = AGG2_CHUNKS

        pltpu.async_copy(g_h.at[src_v.at[0]], buf0, gsems[0])

        def step(j, carry):
            for b in range(2):
                @pl.when(lax.rem(j, 2) == b)
                def _():
                    bn = (b + 1) % 2
                    @pl.when(j + 1 < n)
                    def _():
                        @pl.when(j >= 1)
                        def _():
                            pltpu.make_async_copy(
                                bufs[bn], acc.at[dst_v.at[j]],
                                ssems[bn]).wait()

                        pltpu.async_copy(g_h.at[src_v.at[j + 1]],
                                         bufs[bn], gsems[bn])

                    pltpu.make_async_copy(
                        g_h.at[src_v.at[j]], bufs[b], gsems[b]).wait()
                    pltpu.async_copy(bufs[b], acc.at[dst_v.at[j]],
                                     ssems[b], add=True)

            return carry

        lax.fori_loop(0, n, step, 0)
        for t in range(2):
            j = n - 1 - t
            pltpu.make_async_copy(bufs[j % 2], acc.at[dst_v.at[0]],
                                  ssems[j % 2]).wait()
        plsc.subcore_barrier()
        pltpu.sync_copy(acc.at[pl.ds(r0, ROWS_PER_TILE)],
                        out.at[cid, pl.ds(r0, ROWS_PER_TILE)])

    kfn = pl.kernel(
        body,
        out_type=jax.ShapeDtypeStruct((N_SC, NPAD, 64), jnp.float32),
        mesh=_sc_mesh(),
        scratch_types=[
            pltpu.VMEM((AGG2_CHUNKS, CHUNK), jnp.int32),
            pltpu.VMEM((AGG2_CHUNKS, CHUNK), jnp.int32),
            pltpu.VMEM((CHUNK, 64), jnp.float32),
            pltpu.VMEM((CHUNK, 64), jnp.float32),
            pltpu.VMEM_SHARED((NPAD, 64), jnp.float32),
            pltpu.SemaphoreType.DMA,
            pltpu.SemaphoreType.DMA,
            pltpu.SemaphoreType.DMA,
            pltpu.SemaphoreType.DMA,
        ],
        compiler_params=_SC_PARAMS,
    )
    return kfn(g2, srcs, dsts, zeros_h)


def _mm1(x_pad, W1):
    def body(x_ref, w_ref, o_ref):
        o_ref[...] = jnp.dot(x_ref[...], w_ref[...],
                             preferred_element_type=jnp.float32)

    return pl.pallas_call(
        body,
        grid=(NPAD // 512,),
        in_specs=[pl.BlockSpec((512, 256), lambda i: (i, 0)),
                  pl.BlockSpec((256, 256), lambda i: (0, 0))],
        out_specs=pl.BlockSpec((512, 256), lambda i: (i, 0)),
        out_shape=jax.ShapeDtypeStruct((NPAD, 256), jnp.float32),
    )(x_pad, W1)


def _dinv_of(d_blk):
    deg = d_blk[0, :, 0:1] + d_blk[1, :, 0:1] + 1.0
    return lax.rsqrt(deg)


def _scale_split(h1, deg_parts):
    bm = 256

    def body(h_ref, d_ref, g_ref, dv_ref):
        dinv = _dinv_of(d_ref)
        g = h_ref[...] * dinv
        g_ref[0] = g[:, :128]
        g_ref[1] = g[:, 128:]
        dv_ref[...] = jnp.broadcast_to(dinv, (bm, 128))

    return pl.pallas_call(
        body,
        grid=(NPAD // bm,),
        in_specs=[pl.BlockSpec((bm, 256), lambda i: (i, 0)),
                  pl.BlockSpec((2, bm, 16), lambda i: (0, i, 0))],
        out_specs=[pl.BlockSpec((2, bm, 128), lambda i: (0, i, 0)),
                   pl.BlockSpec((bm, 128), lambda i: (i, 0))],
        out_shape=[jax.ShapeDtypeStruct((2, NPAD, 128), jnp.float32),
                   jax.ShapeDtypeStruct((NPAD, 128), jnp.float32)],
    )(h1, deg_parts)


def _layer2(agg1, dinv_b, b1r, W2):
    bm = 256

    def body(a_ref, dv_ref, b1_ref, w2_ref, g2_ref):
        dinv = dv_ref[:, 0:1]
        z1 = jnp.concatenate([a_ref[0], a_ref[1]], axis=1)
        z1 = jax.nn.relu(z1 * dinv + b1_ref[...])
        h2 = jnp.dot(z1, w2_ref[...], preferred_element_type=jnp.float32)
        g2 = h2 * dinv
        g2_ref[0] = g2[:, :32]
        g2_ref[1] = g2[:, 32:]

    return pl.pallas_call(
        body,
        grid=(NPAD // bm,),
        in_specs=[pl.BlockSpec((2, bm, 128), lambda i: (0, i, 0)),
                  pl.BlockSpec((bm, 128), lambda i: (i, 0)),
                  pl.BlockSpec((1, 256), lambda i: (0, 0)),
                  pl.BlockSpec((256, 64), lambda i: (0, 0))],
        out_specs=pl.BlockSpec((2, bm, 32), lambda i: (0, i, 0)),
        out_shape=jax.ShapeDtypeStruct((2, NPAD, 32), jnp.float32),
    )(agg1, dinv_b, b1r, W2)


def _zout(agg2, dinv_b, b2r):
    bm = 256

    def body(a_ref, dv_ref, b2_ref, z_ref):
        dinv = dv_ref[:, 0:1]
        zc = jnp.concatenate([a_ref[0], a_ref[1]], axis=1)
        z_ref[...] = zc * dinv + b2_ref[...]

    return pl.pallas_call(
        body,
        grid=(NPAD // bm,),
        in_specs=[pl.BlockSpec((2, bm, 32), lambda i: (0, i, 0)),
                  pl.BlockSpec((bm, 128), lambda i: (i, 0)),
                  pl.BlockSpec((1, 64), lambda i: (0, 0))],
        out_specs=pl.BlockSpec((bm, 64), lambda i: (i, 0)),
        out_shape=jax.ShapeDtypeStruct((NPAD, 64), jnp.float32),
    )(agg2, dinv_b, b2r)


def _decoder(z_pad):
    bm = 1024

    def body(a_ref, b_ref, o_ref):
        prod = lax.dot_general(a_ref[...], b_ref[...],
                               (((1,), (1,)), ((), ())),
                               preferred_element_type=jnp.float32)
        o_ref[...] = jax.nn.sigmoid(prod)

    return pl.pallas_call(
        body,
        grid=(NPAD // bm, NPAD // bm),
        in_specs=[pl.BlockSpec((bm, 64), lambda i, j: (i, 0)),
                  pl.BlockSpec((bm, 64), lambda i, j: (j, 0))],
        out_specs=pl.BlockSpec((bm, bm), lambda i, j: (i, j)),
        out_shape=jax.ShapeDtypeStruct((N, N), jnp.float32),
    )(z_pad, z_pad)


def kernel(x, edge_index, W1, b1, W2, b2):
    src = edge_index[0].astype(jnp.int32)
    dst = edge_index[1].astype(jnp.int32)
    pad = EPAD - E
    src_p = jnp.concatenate([src, jnp.full((pad,), N, jnp.int32)])
    dst_p = jnp.concatenate([dst, jnp.full((pad,), N, jnp.int32)])

    dst_deg = dst_p.reshape(N_SC, N_TILE, DEG_CHUNKS, CHUNK)
    src_t = src_p.reshape(N_TILE, AGG_CHUNKS, CHUNK)
    srcs_agg = jnp.stack([src_t, src_t + NPAD])
    dst_agg = dst_p.reshape(N_TILE, AGG_CHUNKS, CHUNK)
    srcs2 = src_p.reshape(N_SC, N_TILE, AGG2_CHUNKS, CHUNK)
    dsts2 = dst_p.reshape(N_SC, N_TILE, AGG2_CHUNKS, CHUNK)

    x_pad = jnp.pad(x, ((0, NPAD - N), (0, 0)))
    ones_h = jnp.ones((CHUNK, 16), jnp.float32)
    zeros_h = jnp.zeros((ROWS_PER_TILE, 16), jnp.float32)
    zeros64_h = jnp.zeros((ROWS_PER_TILE, 64), jnp.float32)

    deg_parts = _deg(dst_deg, ones_h, zeros_h)
    h1 = _mm1(x_pad, W1)
    g1_cat, dinv_b = _scale_split(h1, deg_parts)
    agg1 = _agg(g1_cat.reshape(N_SC * NPAD, 128), srcs_agg, dst_agg, 128)
    g2_cat = _layer2(agg1, dinv_b, b1.reshape(1, 256), W2)
    agg2 = _agg(g2_cat.reshape(N_SC * NPAD, 32), srcs_agg, dst_agg, 32)
    z_pad = _zout(agg2, dinv_b, b2.reshape(1, 64))
    recons = _decoder(z_pad)
    return (recons, z_pad[:N])


# speedup vs baseline: 1.0790x; 1.0207x over previous
import functools

import jax
import jax.numpy as jnp
from jax import lax
from jax.experimental import pallas as pl
from jax.experimental.pallas import tpu as pltpu
from jax.experimental.pallas import tpu_sc as plsc

N = 10000
NPAD = 10240
E = 160000
EPAD = 163840
CHUNK = 128
N_SC = 2
N_TILE = 16
ROWS_PER_TILE = NPAD // N_TILE
DEG_CHUNKS = EPAD // (N_SC * N_TILE * CHUNK)
AGG_CHUNKS = EPAD // (N_TILE * CHUNK)


def _sc_mesh():
    return plsc.VectorSubcoreMesh(core_axis_name="c", subcore_axis_name="s")


_SC_PARAMS = pltpu.CompilerParams(use_tc_tiling_on_sc=False)


def _deg(dsts, ones_h, zeros_h):
    def body(dst_h, ones_hbm, zeros_hbm, out, ones_v, dst_v, acc):
        cid = lax.axis_index("c")
        sid = lax.axis_index("s")
        pltpu.sync_copy(dst_h.at[cid, sid], dst_v)
        pltpu.sync_copy(ones_hbm, ones_v)
        r0 = sid * ROWS_PER_TILE
        pltpu.sync_copy(zeros_hbm, acc.at[pl.ds(r0, ROWS_PER_TILE)])
        plsc.subcore_barrier()

        def step(j, carry):
            pltpu.sync_copy(ones_v, acc.at[dst_v.at[j]], add=True)
            return carry

        lax.fori_loop(0, DEG_CHUNKS, step, 0)
        plsc.subcore_barrier()
        pltpu.sync_copy(acc.at[pl.ds(r0, ROWS_PER_TILE)],
                        out.at[cid, pl.ds(r0, ROWS_PER_TILE)])

    kfn = pl.kernel(
        body,
        out_type=jax.ShapeDtypeStruct((N_SC, NPAD, 16), jnp.float32),
        mesh=_sc_mesh(),
        scratch_types=[
            pltpu.VMEM((CHUNK, 16), jnp.float32),
            pltpu.VMEM((DEG_CHUNKS, CHUNK), jnp.int32),
            pltpu.VMEM_SHARED((NPAD, 16), jnp.float32),
        ],
        compiler_params=_SC_PARAMS,
    )
    return kfn(dsts, ones_h, zeros_h)


def _agg(g_cat, srcs, dsts, feat):
    n_pass = 2 if feat > 64 else 1
    chunks_per_pass = AGG_CHUNKS // n_pass

    def body(g_h, src_h, dst_h, out, src_v, dst_v, buf0, buf1, acc,
             gs0, gs1, ss0, ss1):
        cid = lax.axis_index("c")
        sid = lax.axis_index("s")
        r0 = sid * ROWS_PER_TILE
        pltpu.sync_copy(g_h.at[pl.ds(cid * NPAD + r0, ROWS_PER_TILE)],
                        acc.at[pl.ds(r0, ROWS_PER_TILE)])
        plsc.subcore_barrier()

        bufs = (buf0, buf1)
        gsems = (gs0, gs1)
        ssems = (ss0, ss1)
        n = chunks_per_pass

        for p in range(n_pass):
            c0 = p * n
            pltpu.sync_copy(src_h.at[cid, sid, pl.ds(c0, n)], src_v)
            pltpu.sync_copy(dst_h.at[sid, pl.ds(c0, n)], dst_v)

            pltpu.async_copy(g_h.at[src_v.at[0]], buf0, gsems[0])

            def step(j, carry):
                for b in range(2):
                    @pl.when(lax.rem(j, 2) == b)
                    def _():
                        bn = (b + 1) % 2
                        @pl.when(j + 1 < n)
                        def _():
                            @pl.when(j >= 1)
                            def _():
                                pltpu.make_async_copy(
                                    bufs[bn], acc.at[dst_v.at[j]],
                                    ssems[bn]).wait()

                            pltpu.async_copy(g_h.at[src_v.at[j + 1]],
                                             bufs[bn], gsems[bn])

                        pltpu.make_async_copy(
                            g_h.at[src_v.at[j]], bufs[b], gsems[b]).wait()
                        pltpu.async_copy(bufs[b], acc.at[dst_v.at[j]],
                                         ssems[b], add=True)

                return carry

            lax.fori_loop(0, n, step, 0)
            for t in range(2):
                j = n - 1 - t
                pltpu.make_async_copy(bufs[j % 2], acc.at[dst_v.at[0]],
                                      ssems[j % 2]).wait()
        plsc.subcore_barrier()
        pltpu.sync_copy(acc.at[pl.ds(r0, ROWS_PER_TILE)],
                        out.at[cid, pl.ds(r0, ROWS_PER_TILE)])

    kfn = pl.kernel(
        body,
        out_type=jax.ShapeDtypeStruct((N_SC, NPAD, feat), jnp.float32),
        mesh=_sc_mesh(),
        scratch_types=[
            pltpu.VMEM((AGG_CHUNKS // n_pass, CHUNK), jnp.int32),
            pltpu.VMEM((AGG_CHUNKS // n_pass, CHUNK), jnp.int32),
            pltpu.VMEM((CHUNK, feat), jnp.float32),
            pltpu.VMEM((CHUNK, feat), jnp.float32),
            pltpu.VMEM_SHARED((NPAD, feat), jnp.float32),
            pltpu.SemaphoreType.DMA,
            pltpu.SemaphoreType.DMA,
            pltpu.SemaphoreType.DMA,
            pltpu.SemaphoreType.DMA,
        ],
        compiler_params=_SC_PARAMS,
    )
    return kfn(g_cat, srcs, dsts)


def _dinv_of(d_blk):
    deg = d_blk[0, :, 0:1] + d_blk[1, :, 0:1] + 1.0
    return lax.rsqrt(deg)


def _mm1_scale(x, W1, deg_parts):
    bm = 512
    ni = NPAD // bm

    def body(x_ref, w_ref, d_ref, g_ref, dv_ref):
        dinv = _dinv_of(d_ref)
        h = jnp.dot(x_ref[...], w_ref[...], preferred_element_type=jnp.float32)
        g_ref[...] = h * dinv
        dv_ref[...] = jnp.broadcast_to(dinv, (bm, 128))

    return pl.pallas_call(
        body,
        grid=(ni, 2),
        in_specs=[pl.BlockSpec((bm, 256), lambda i, c: (i, 0)),
                  pl.BlockSpec((256, 128), lambda i, c: (0, c)),
                  pl.BlockSpec((2, bm, 16), lambda i, c: (0, i, 0))],
        out_specs=[pl.BlockSpec((bm, 128), lambda i, c: (c * ni + i, 0)),
                   pl.BlockSpec((bm, 128), lambda i, c: (i, 0))],
        out_shape=[jax.ShapeDtypeStruct((2 * NPAD, 128), jnp.float32),
                   jax.ShapeDtypeStruct((NPAD, 128), jnp.float32)],
    )(x, W1, deg_parts)


def _layer2(agg1, dinv_b, b1r, W2):
    bm = 512

    def body(a_ref, dv_ref, b1_ref, w2_ref, g2_ref):
        dinv = dv_ref[:, 0:1]
        z1 = jnp.concatenate([a_ref[0], a_ref[1]], axis=1)
        z1 = jax.nn.relu(z1 * dinv + b1_ref[...])
        h2 = jnp.dot(z1, w2_ref[...], preferred_element_type=jnp.float32)
        g2 = h2 * dinv
        g2_ref[0] = g2[:, :32]
        g2_ref[1] = g2[:, 32:]

    return pl.pallas_call(
        body,
        grid=(NPAD // bm,),
        in_specs=[pl.BlockSpec((2, bm, 128), lambda i: (0, i, 0)),
                  pl.BlockSpec((bm, 128), lambda i: (i, 0)),
                  pl.BlockSpec((1, 256), lambda i: (0, 0)),
                  pl.BlockSpec((256, 64), lambda i: (0, 0))],
        out_specs=pl.BlockSpec((2, bm, 32), lambda i: (0, i, 0)),
        out_shape=jax.ShapeDtypeStruct((2, NPAD, 32), jnp.float32),
    )(agg1, dinv_b, b1r, W2)


def _zout(agg2, dinv_b, b2r):
    bm = 256

    def body(a_ref, dv_ref, b2_ref, z_ref):
        dinv = dv_ref[:, 0:1]
        zc = jnp.concatenate([a_ref[0], a_ref[1]], axis=1)
        z_ref[...] = zc * dinv + b2_ref[...]

    return pl.pallas_call(
        body,
        grid=(NPAD // bm,),
        in_specs=[pl.BlockSpec((2, bm, 32), lambda i: (0, i, 0)),
                  pl.BlockSpec((bm, 128), lambda i: (i, 0)),
                  pl.BlockSpec((1, 64), lambda i: (0, 0))],
        out_specs=pl.BlockSpec((bm, 64), lambda i: (i, 0)),
        out_shape=jax.ShapeDtypeStruct((NPAD, 64), jnp.float32),
    )(agg2, dinv_b, b2r)


def _decoder(z_pad):
    bm = 1024

    def body(a_ref, b_ref, o_ref):
        prod = lax.dot_general(a_ref[...], b_ref[...],
                               (((1,), (1,)), ((), ())),
                               preferred_element_type=jnp.float32)
        o_ref[...] = jax.nn.sigmoid(prod)

    return pl.pallas_call(
        body,
        grid=(NPAD // bm, NPAD // bm),
        in_specs=[pl.BlockSpec((bm, 64), lambda i, j: (i, 0)),
                  pl.BlockSpec((bm, 64), lambda i, j: (j, 0))],
        out_specs=pl.BlockSpec((bm, bm), lambda i, j: (i, j)),
        out_shape=jax.ShapeDtypeStruct((N, N), jnp.float32),
    )(z_pad, z_pad)


def kernel(x, edge_index, W1, b1, W2, b2):
    src = edge_index[0].astype(jnp.int32)
    dst = edge_index[1].astype(jnp.int32)
    pad = EPAD - E
    src_p = jnp.concatenate([src, jnp.full((pad,), N, jnp.int32)])
    dst_p = jnp.concatenate([dst, jnp.full((pad,), N, jnp.int32)])

    dst_deg = dst_p.reshape(N_SC, N_TILE, DEG_CHUNKS, CHUNK)
    src_t = src_p.reshape(N_TILE, AGG_CHUNKS, CHUNK)
    srcs_agg = jnp.stack([src_t, src_t + NPAD])
    dst_agg = dst_p.reshape(N_TILE, AGG_CHUNKS, CHUNK)
    ones_h = jnp.ones((CHUNK, 16), jnp.float32)
    zeros_h = jnp.zeros((ROWS_PER_TILE, 16), jnp.float32)

    x_pad = jnp.pad(x, ((0, NPAD - N), (0, 0)))
    deg_parts = _deg(dst_deg, ones_h, zeros_h)
    g1_flat, dinv_b = _mm1_scale(x_pad, W1, deg_parts)
    agg1 = _agg(g1_flat, srcs_agg, dst_agg, 128)
    g2_cat = _layer2(agg1, dinv_b, b1.reshape(1, 256), W2)
    agg2 = _agg(g2_cat.reshape(N_SC * NPAD, 32), srcs_agg, dst_agg, 32)
    z_pad = _zout(agg2, dinv_b, b2.reshape(1, 64))
    recons = _decoder(z_pad)
    return (recons, z_pad[:N])


# speedup vs baseline: 1.1471x; 1.0632x over previous
import functools

import jax
import jax.numpy as jnp
from jax import lax
from jax.experimental import pallas as pl
from jax.experimental.pallas import tpu as pltpu
from jax.experimental.pallas import tpu_sc as plsc

N = 10000
NPAD = 10240
E = 160000
EPAD = 163840
CHUNK = 128
N_SC = 2
N_TILE = 16
ROWS_PER_TILE = NPAD // N_TILE
DEG_CHUNKS = EPAD // (N_SC * N_TILE * CHUNK)
AGG_CHUNKS = EPAD // (N_TILE * CHUNK)


def _sc_mesh():
    return plsc.VectorSubcoreMesh(core_axis_name="c", subcore_axis_name="s")


_SC_PARAMS = pltpu.CompilerParams(use_tc_tiling_on_sc=False)


def _deg(dsts, ones_h, zeros_h):
    def body(dst_h, ones_hbm, zeros_hbm, out, ones_v, dst_v, acc):
        cid = lax.axis_index("c")
        sid = lax.axis_index("s")
        pltpu.sync_copy(dst_h.at[cid, sid], dst_v)
        pltpu.sync_copy(ones_hbm, ones_v)
        r0 = sid * ROWS_PER_TILE
        pltpu.sync_copy(zeros_hbm, acc.at[pl.ds(r0, ROWS_PER_TILE)])
        plsc.subcore_barrier()

        def step(j, carry):
            pltpu.sync_copy(ones_v, acc.at[dst_v.at[j]], add=True)
            return carry

        lax.fori_loop(0, DEG_CHUNKS, step, 0)
        plsc.subcore_barrier()
        pltpu.sync_copy(acc.at[pl.ds(r0, ROWS_PER_TILE)],
                        out.at[cid, pl.ds(r0, ROWS_PER_TILE)])

    kfn = pl.kernel(
        body,
        out_type=jax.ShapeDtypeStruct((N_SC, NPAD, 16), jnp.float32),
        mesh=_sc_mesh(),
        scratch_types=[
            pltpu.VMEM((CHUNK, 16), jnp.float32),
            pltpu.VMEM((DEG_CHUNKS, CHUNK), jnp.int32),
            pltpu.VMEM_SHARED((NPAD, 16), jnp.float32),
        ],
        compiler_params=_SC_PARAMS,
    )
    return kfn(dsts, ones_h, zeros_h)


def _agg(g_cat, srcs, dsts, feat):
    n_pass = 2 if feat > 64 else 1
    chunks_per_pass = AGG_CHUNKS // n_pass

    def body(g_h, src_h, dst_h, out, src_v, dst_v, buf0, buf1, acc,
             gs0, gs1, ss0, ss1):
        cid = lax.axis_index("c")
        sid = lax.axis_index("s")
        r0 = sid * ROWS_PER_TILE
        pltpu.sync_copy(g_h.at[pl.ds(cid * NPAD + r0, ROWS_PER_TILE)],
                        acc.at[pl.ds(r0, ROWS_PER_TILE)])
        plsc.subcore_barrier()

        bufs = (buf0, buf1)
        gsems = (gs0, gs1)
        ssems = (ss0, ss1)
        n = chunks_per_pass

        for p in range(n_pass):
            c0 = p * n
            pltpu.sync_copy(src_h.at[cid, sid, pl.ds(c0, n)], src_v)
            pltpu.sync_copy(dst_h.at[sid, pl.ds(c0, n)], dst_v)

            pltpu.async_copy(g_h.at[src_v.at[0]], buf0, gsems[0])

            def step(j, carry):
                for b in range(2):
                    @pl.when(lax.rem(j, 2) == b)
                    def _():
                        bn = (b + 1) % 2
                        @pl.when(j + 1 < n)
                        def _():
                            @pl.when(j >= 1)
                            def _():
                                pltpu.make_async_copy(
                                    bufs[bn], acc.at[dst_v.at[j]],
                                    ssems[bn]).wait()

                            pltpu.async_copy(g_h.at[src_v.at[j + 1]],
                                             bufs[bn], gsems[bn])

                        pltpu.make_async_copy(
                            g_h.at[src_v.at[j]], bufs[b], gsems[b]).wait()
                        pltpu.async_copy(bufs[b], acc.at[dst_v.at[j]],
                                         ssems[b], add=True)

                return carry

            lax.fori_loop(0, n, step, 0)
            for t in range(2):
                j = n - 1 - t
                pltpu.make_async_copy(bufs[j % 2], acc.at[dst_v.at[0]],
                                      ssems[j % 2]).wait()
        plsc.subcore_barrier()
        pltpu.sync_copy(acc.at[pl.ds(r0, ROWS_PER_TILE)],
                        out.at[cid, pl.ds(r0, ROWS_PER_TILE)])

    kfn = pl.kernel(
        body,
        out_type=jax.ShapeDtypeStruct((N_SC, NPAD, feat), jnp.float32),
        mesh=_sc_mesh(),
        scratch_types=[
            pltpu.VMEM((AGG_CHUNKS // n_pass, CHUNK), jnp.int32),
            pltpu.VMEM((AGG_CHUNKS // n_pass, CHUNK), jnp.int32),
            pltpu.VMEM((CHUNK, feat), jnp.float32),
            pltpu.VMEM((CHUNK, feat), jnp.float32),
            pltpu.VMEM_SHARED((NPAD, feat), jnp.float32),
            pltpu.SemaphoreType.DMA,
            pltpu.SemaphoreType.DMA,
            pltpu.SemaphoreType.DMA,
            pltpu.SemaphoreType.DMA,
        ],
        compiler_params=_SC_PARAMS,
    )
    return kfn(g_cat, srcs, dsts)


def _dinv_of(d_blk):
    deg = d_blk[0, :, 0:1] + d_blk[1, :, 0:1] + 1.0
    return lax.rsqrt(deg)


def _mm1_scale(x, W1, deg_parts):
    bm = 512
    ni = NPAD // bm

    def body(x_ref, w_ref, d_ref, g_ref, dv_ref):
        dinv = _dinv_of(d_ref)
        h = jnp.dot(x_ref[...], w_ref[...], preferred_element_type=jnp.float32)
        g_ref[...] = h * dinv
        dv_ref[...] = jnp.broadcast_to(dinv, (bm, 128))

    return pl.pallas_call(
        body,
        grid=(ni, 2),
        in_specs=[pl.BlockSpec((bm, 256), lambda i, c: (i, 0)),
                  pl.BlockSpec((256, 128), lambda i, c: (0, c)),
                  pl.BlockSpec((2, bm, 16), lambda i, c: (0, i, 0))],
        out_specs=[pl.BlockSpec((bm, 128), lambda i, c: (c * ni + i, 0)),
                   pl.BlockSpec((bm, 128), lambda i, c: (i, 0))],
        out_shape=[jax.ShapeDtypeStruct((2 * NPAD, 128), jnp.float32),
                   jax.ShapeDtypeStruct((NPAD, 128), jnp.float32)],
    )(x, W1, deg_parts)


def _layer2(agg1, dinv_b, b1r, W2):
    bm = 512

    def body(a_ref, dv_ref, b1_ref, w2_ref, g2_ref):
        dinv = dv_ref[:, 0:1]
        z1 = jnp.concatenate([a_ref[0], a_ref[1]], axis=1)
        z1 = jax.nn.relu(z1 * dinv + b1_ref[...])
        h2 = jnp.dot(z1, w2_ref[...], preferred_element_type=jnp.float32)
        g2 = h2 * dinv
        g2_ref[0] = g2[:, :32]
        g2_ref[1] = g2[:, 32:]

    return pl.pallas_call(
        body,
        grid=(NPAD // bm,),
        in_specs=[pl.BlockSpec((2, bm, 128), lambda i: (0, i, 0)),
                  pl.BlockSpec((bm, 128), lambda i: (i, 0)),
                  pl.BlockSpec((1, 256), lambda i: (0, 0)),
                  pl.BlockSpec((256, 64), lambda i: (0, 0))],
        out_specs=pl.BlockSpec((2, bm, 32), lambda i: (0, i, 0)),
        out_shape=jax.ShapeDtypeStruct((2, NPAD, 32), jnp.float32),
    )(agg1, dinv_b, b1r, W2)


def _zout(agg2, dinv_b, b2r):
    bm = 256

    def body(a_ref, dv_ref, b2_ref, z_ref):
        dinv = dv_ref[:, 0:1]
        zc = jnp.concatenate([a_ref[0], a_ref[1]], axis=1)
        z_ref[...] = zc * dinv + b2_ref[...]

    return pl.pallas_call(
        body,
        grid=(NPAD // bm,),
        in_specs=[pl.BlockSpec((2, bm, 32), lambda i: (0, i, 0)),
                  pl.BlockSpec((bm, 128), lambda i: (i, 0)),
                  pl.BlockSpec((1, 64), lambda i: (0, 0))],
        out_specs=pl.BlockSpec((bm, 64), lambda i: (i, 0)),
        out_shape=jax.ShapeDtypeStruct((NPAD, 64), jnp.float32),
    )(agg2, dinv_b, b2r)


def _decoder(z_pad):
    bm = 2048

    def body(a_ref, b_ref, o_ref):
        prod = lax.dot_general(a_ref[...], b_ref[...],
                               (((1,), (1,)), ((), ())),
                               preferred_element_type=jnp.float32)
        o_ref[...] = jax.nn.sigmoid(prod)

    return pl.pallas_call(
        body,
        grid=(NPAD // bm, NPAD // bm),
        in_specs=[pl.BlockSpec((bm, 64), lambda i, j: (i, 0)),
                  pl.BlockSpec((bm, 64), lambda i, j: (j, 0))],
        out_specs=pl.BlockSpec((bm, bm), lambda i, j: (i, j)),
        out_shape=jax.ShapeDtypeStruct((N, N), jnp.float32),
    )(z_pad, z_pad)


def kernel(x, edge_index, W1, b1, W2, b2):
    src = edge_index[0].astype(jnp.int32)
    dst = edge_index[1].astype(jnp.int32)
    pad = EPAD - E
    src_p = jnp.concatenate([src, jnp.full((pad,), N, jnp.int32)])
    dst_p = jnp.concatenate([dst, jnp.full((pad,), N, jnp.int32)])

    dst_deg = dst_p.reshape(N_SC, N_TILE, DEG_CHUNKS, CHUNK)
    src_t = src_p.reshape(N_TILE, AGG_CHUNKS, CHUNK)
    srcs_agg = jnp.stack([src_t, src_t + NPAD])
    dst_agg = dst_p.reshape(N_TILE, AGG_CHUNKS, CHUNK)
    ones_h = jnp.ones((CHUNK, 16), jnp.float32)
    zeros_h = jnp.zeros((ROWS_PER_TILE, 16), jnp.float32)

    x_pad = jnp.pad(x, ((0, NPAD - N), (0, 0)))
    deg_parts = _deg(dst_deg, ones_h, zeros_h)
    g1_flat, dinv_b = _mm1_scale(x_pad, W1, deg_parts)
    agg1 = _agg(g1_flat, srcs_agg, dst_agg, 128)
    g2_cat = _layer2(agg1, dinv_b, b1.reshape(1, 256), W2)
    agg2 = _agg(g2_cat.reshape(N_SC * NPAD, 32), srcs_agg, dst_agg, 32)
    z_pad = _zout(agg2, dinv_b, b2.reshape(1, 64))
    recons = _decoder(z_pad)
    return (recons, z_pad[:N])
